# baseline (speedup 1.0000x reference)
import functools

import jax
import jax.numpy as jnp
from jax.experimental import pallas as pl

_N = 10000
_E = 320000
_NG = 64

_TILE_N = 1000
_TILE_E = 8000


def _proj_body(x_ref, w_ref, asrc_ref, adst_ref, xp_ref, as_ref, ad_ref):
    xp = jnp.dot(x_ref[...], w_ref[...], preferred_element_type=jnp.float32)
    xp_ref[...] = xp
    as_ref[...] = jnp.dot(xp, asrc_ref[...], preferred_element_type=jnp.float32)
    ad_ref[...] = jnp.dot(xp, adst_ref[...], preferred_element_type=jnp.float32)


def _proj(x, W, asrc_mat, adst_mat):
    n, f = x.shape
    hc = W.shape[1]
    h = asrc_mat.shape[1]
    grid = (n // _TILE_N,)
    return pl.pallas_call(
        _proj_body,
        grid=grid,
        in_specs=[
            pl.BlockSpec((_TILE_N, f), lambda i: (i, 0)),
            pl.BlockSpec((f, hc), lambda i: (0, 0)),
            pl.BlockSpec((hc, h), lambda i: (0, 0)),
            pl.BlockSpec((hc, h), lambda i: (0, 0)),
        ],
        out_specs=[
            pl.BlockSpec((_TILE_N, hc), lambda i: (i, 0)),
            pl.BlockSpec((_TILE_N, h), lambda i: (i, 0)),
            pl.BlockSpec((_TILE_N, h), lambda i: (i, 0)),
        ],
        out_shape=[
            jax.ShapeDtypeStruct((n, hc), jnp.float32),
            jax.ShapeDtypeStruct((n, h), jnp.float32),
            jax.ShapeDtypeStruct((n, h), jnp.float32),
        ],
    )(x, W, asrc_mat, adst_mat)


def _alpha_body(a_ref, b_ref, o_ref):
    s = a_ref[...] + b_ref[...]
    o_ref[...] = jnp.where(s >= 0.0, s, 0.2 * s)


def _ex_body(al_ref, mx_ref, o_ref):
    o_ref[...] = jnp.exp(al_ref[...] - mx_ref[...])


def _norm_body(ex_ref, dn_ref, o_ref):
    o_ref[...] = ex_ref[...] / (dn_ref[...] + 1e-16)


def _edge_ew(body, a, b):
    e, h = a.shape
    grid = (e // _TILE_E,)
    spec = pl.BlockSpec((_TILE_E, h), lambda i: (i, 0))
    return pl.pallas_call(
        body,
        grid=grid,
        in_specs=[spec, spec],
        out_specs=spec,
        out_shape=jax.ShapeDtypeStruct((e, h), jnp.float32),
    )(a, b)


def _bias_elu_body(x_ref, b_ref, o_ref):
    y = x_ref[...] + b_ref[...]
    o_ref[...] = jnp.where(y > 0.0, y, jnp.exp(y) - 1.0)


def _bias_elu(x, b):
    n, d = x.shape
    grid = (n // _TILE_N,)
    return pl.pallas_call(
        _bias_elu_body,
        grid=grid,
        in_specs=[
            pl.BlockSpec((_TILE_N, d), lambda i: (i, 0)),
            pl.BlockSpec((1, d), lambda i: (0, 0)),
        ],
        out_specs=pl.BlockSpec((_TILE_N, d), lambda i: (i, 0)),
        out_shape=jax.ShapeDtypeStruct((n, d), jnp.float32),
    )(x, b.reshape(1, d))


def _att_mat(att):
    heads, ch = att.shape
    eye = jnp.eye(heads, dtype=jnp.float32)
    return (att[:, :, None] * eye[:, None, :]).reshape(heads * ch, heads)


def _gat_layer(x, src, dst, W, att_src, att_dst, b, heads, ch):
    n = x.shape[0]
    xp, a_src, a_dst = _proj(x, W, _att_mat(att_src), _att_mat(att_dst))
    alpha = _edge_ew(_alpha_body, jnp.take(a_src, src, axis=0),
                     jnp.take(a_dst, dst, axis=0))
    amax = jax.ops.segment_max(alpha, dst, num_segments=n)
    amax = jnp.where(jnp.isfinite(amax), amax, 0.0)
    ex = _edge_ew(_ex_body, alpha, jnp.take(amax, dst, axis=0))
    denom = jax.ops.segment_sum(ex, dst, num_segments=n)
    alpha_n = _edge_ew(_norm_body, ex, jnp.take(denom, dst, axis=0))
    msg = jnp.take(xp, src, axis=0).reshape(-1, heads, ch) * alpha_n[:, :, None]
    out = jax.ops.segment_sum(msg, dst, num_segments=n).reshape(n, heads * ch)
    return _bias_elu(out, b), alpha_n


def kernel(x1, edge_index, batch, W1, att_src1, att_dst1, b1, W2, att_src2, att_dst2, b2):
    src = edge_index[0]
    dst = edge_index[1]
    h1 = W1.shape[1] // 128
    h, alpha1 = _gat_layer(x1, src, dst, W1, att_src1, att_dst1, b1,
                           att_src1.shape[0], att_src1.shape[1])
    h, _ = _gat_layer(h, src, dst, W2, att_src2, att_dst2, b2,
                      att_src2.shape[0], att_src2.shape[1])
    pooled = jax.ops.segment_max(h, batch, num_segments=_NG)
    return (pooled, alpha1)
